# baseline (speedup 1.0000x reference)
import functools

import jax
import jax.numpy as jnp
from jax.experimental import pallas as pl

N_PIXELS = 147456
C = 64
N_SEEDS = 128
TP = 2048


def _body(fa_ref, fs_ref, sim_ref, blobs_ref):
    fa = fa_ref[...]
    fs = fs_ref[...]
    cross = jax.lax.dot_general(
        fa, fs, (((0,), (0,)), ((), ())),
        preferred_element_type=jnp.float32)
    a2 = jnp.sum(fa * fa, axis=0)
    s2 = jnp.sum(fs * fs, axis=0)
    arg = cross * 0.03125 - (a2[:, None] * 0.015625 + s2[None, :] * 0.015625)
    sim = jnp.clip(jnp.exp(arg), 1e-6, 1.0 - 1e-6)
    sim_ref[0] = sim
    mx = jnp.max(sim, axis=1, keepdims=True)
    idx = jax.lax.broadcasted_iota(jnp.int32, sim.shape, 1)
    blobs_ref[0] = jnp.min(jnp.where(sim == mx, idx, N_SEEDS), axis=1)


@functools.partial(jax.jit, static_argnames=())
def kernel(fA, fS):
    fa = fA[0]
    fs = fS[0]
    grid = (N_PIXELS // TP,)
    sim, blobs = pl.pallas_call(
        _body,
        grid=grid,
        in_specs=[
            pl.BlockSpec((C, TP), lambda i: (0, i)),
            pl.BlockSpec((C, N_SEEDS), lambda i: (0, 0)),
        ],
        out_specs=[
            pl.BlockSpec((1, TP, N_SEEDS), lambda i: (0, i, 0)),
            pl.BlockSpec((1, TP), lambda i: (0, i)),
        ],
        out_shape=[
            jax.ShapeDtypeStruct((1, N_PIXELS, N_SEEDS), jnp.float32),
            jax.ShapeDtypeStruct((1, N_PIXELS), jnp.int32),
        ],
    )(fa, fs)
    return sim, blobs


# speedup vs baseline: 1.5627x; 1.5627x over previous
import functools

import jax
import jax.numpy as jnp
from jax.experimental import pallas as pl

N_PIXELS = 147456
C = 64
N_SEEDS = 128
TP = 2048


def _body(fa_ref, fs_ref, sim_ref, blobs_ref):
    fa = fa_ref[...]
    fs = fs_ref[...]
    cross = jax.lax.dot_general(
        fa, fs, (((0,), (0,)), ((), ())),
        preferred_element_type=jnp.float32)
    a2 = jnp.sum(fa * fa, axis=0)
    s2 = jnp.sum(fs * fs, axis=0)
    arg = cross * 0.03125 - (a2[:, None] * 0.015625 + s2[None, :] * 0.015625)
    sim = jnp.clip(jnp.exp(arg), 1e-6, 1.0 - 1e-6)
    sim_ref[0] = sim
    crossT = jax.lax.dot_general(
        fs, fa, (((0,), (0,)), ((), ())),
        preferred_element_type=jnp.float32)
    argT = crossT * 0.03125 - (s2[:, None] * 0.015625 + a2[None, :] * 0.015625)
    simT = jnp.clip(jnp.exp(argT), 1e-6, 1.0 - 1e-6)
    mxT = jnp.max(simT, axis=0)
    idx0 = jax.lax.broadcasted_iota(jnp.int32, simT.shape, 0)
    blobs_ref[0] = jnp.min(
        jnp.where(simT == mxT[None, :], idx0, N_SEEDS), axis=0)


@functools.partial(jax.jit, static_argnames=())
def kernel(fA, fS):
    fa = fA[0]
    fs = fS[0]
    grid = (N_PIXELS // TP,)
    sim, blobs = pl.pallas_call(
        _body,
        grid=grid,
        in_specs=[
            pl.BlockSpec((C, TP), lambda i: (0, i)),
            pl.BlockSpec((C, N_SEEDS), lambda i: (0, 0)),
        ],
        out_specs=[
            pl.BlockSpec((1, TP, N_SEEDS), lambda i: (0, i, 0)),
            pl.BlockSpec((1, TP), lambda i: (0, i)),
        ],
        out_shape=[
            jax.ShapeDtypeStruct((1, N_PIXELS, N_SEEDS), jnp.float32),
            jax.ShapeDtypeStruct((1, N_PIXELS), jnp.int32),
        ],
    )(fa, fs)
    return sim, blobs
